# Initial kernel scaffold; baseline (speedup 1.0000x reference)
import jax
import jax.numpy as jnp
from jax.experimental import pallas as pl


def kernel(features, adj_matrix, W1, a1, W2, a2, W3, a3, W4, a4, W5, a5):
    raise NotImplementedError("write your pallas kernel here")


# speedup vs baseline: 1.4438x; 1.4438x over previous
import functools

import jax
import jax.numpy as jnp
from jax.experimental import pallas as pl

N = 4096
ALPHA = 0.2
BI = 256


def _prologue_body(x_ref, w_ref, a1_ref, a2_ref, wh_ref, f1_ref, f2_ref,
                   m_ref, *, act):
    x = x_ref[...]
    if act:
        x = jnp.maximum(x, 0.0)
    wh = jnp.dot(x, w_ref[...], preferred_element_type=jnp.float32)
    wh_ref[...] = wh
    f1 = jnp.sum(wh * a1_ref[...], axis=1, keepdims=True)
    f2 = jnp.sum(wh * a2_ref[...], axis=1, keepdims=True)
    f1_ref[...] = f1
    f2_ref[...] = f2
    s = f1 + jnp.max(f2)
    m_ref[...] = jnp.where(s >= 0.0, s, ALPHA * s)


def _attn_body(adj_ref, f1_ref, f2r_ref, m_ref, wh_ref, out_ref):
    s = f1_ref[...] + f2r_ref[...]
    e = jnp.where(s >= 0.0, s, ALPHA * s)
    p = jnp.where(adj_ref[...] > 0.0, jnp.exp(e - m_ref[...]), 0.0)
    denom = jnp.sum(p, axis=1, keepdims=True)
    num = jnp.dot(p, wh_ref[...], preferred_element_type=jnp.float32)
    out_ref[...] = num / denom


def _logsoftmax_body(x_ref, out_ref):
    x = x_ref[...]
    m0 = jnp.max(x, axis=0, keepdims=True)
    lse = jnp.log(jnp.sum(jnp.exp(x - m0), axis=0, keepdims=True)) + m0
    out_ref[...] = x - lse


def _gat_layer(x, adj, W, a, act):
    din, do = W.shape
    a1r = a[:do].reshape(1, do)
    a2r = a[do:].reshape(1, do)
    wh, f1, f2, m = pl.pallas_call(
        functools.partial(_prologue_body, act=act),
        out_shape=[
            jax.ShapeDtypeStruct((N, do), jnp.float32),
            jax.ShapeDtypeStruct((N, 1), jnp.float32),
            jax.ShapeDtypeStruct((N, 1), jnp.float32),
            jax.ShapeDtypeStruct((N, 1), jnp.float32),
        ],
    )(x, W, a1r, a2r)
    f2r = f2.reshape(1, N)
    out = pl.pallas_call(
        _attn_body,
        grid=(N // BI,),
        in_specs=[
            pl.BlockSpec((BI, N), lambda i: (i, 0)),
            pl.BlockSpec((BI, 1), lambda i: (i, 0)),
            pl.BlockSpec((1, N), lambda i: (0, 0)),
            pl.BlockSpec((BI, 1), lambda i: (i, 0)),
            pl.BlockSpec((N, do), lambda i: (0, 0)),
        ],
        out_specs=pl.BlockSpec((BI, do), lambda i: (i, 0)),
        out_shape=jax.ShapeDtypeStruct((N, do), jnp.float32),
    )(adj, f1, f2r, m, wh)
    return out


def kernel(features, adj_matrix, W1, a1, W2, a2, W3, a3, W4, a4, W5, a5):
    x = _gat_layer(features, adj_matrix, W1, a1, act=False)
    x = _gat_layer(x, adj_matrix, W2, a2, act=True)
    x = _gat_layer(x, adj_matrix, W3, a3, act=True)
    x = _gat_layer(x, adj_matrix, W4, a4, act=True)
    x = _gat_layer(x, adj_matrix, W5, a5, act=True)
    out = pl.pallas_call(
        _logsoftmax_body,
        out_shape=jax.ShapeDtypeStruct(x.shape, jnp.float32),
    )(x)
    return out
